# baseline (speedup 1.0000x reference)
import jax
import jax.numpy as jnp
from jax.experimental import pallas as pl
from jax.experimental.pallas import tpu as pltpu

_E = 16
_TOPK = 2

_MB = 512


def _router_body(x_ref, w_ref, probs_ref, idx_ref, wts_ref):
    x = x_ref[...]
    w = w_ref[...]
    logits = jax.lax.dot_general(
        x, w, (((1,), (1,)), ((), ())),
        preferred_element_type=jnp.float32)
    lt = logits.T
    m = jnp.max(lt, axis=0, keepdims=True)
    e = jnp.exp(lt - m)
    s = jnp.sum(e, axis=0, keepdims=True)
    p = e / s
    probs_ref[...] = p

    iota = jax.lax.broadcasted_iota(jnp.int32, p.shape, 0)
    m1 = jnp.max(p, axis=0, keepdims=True)
    c1 = jnp.where(p == m1, iota, _E)
    i1 = jnp.min(c1, axis=0, keepdims=True)
    masked = jnp.where(iota == i1, -1.0, p)
    m2 = jnp.max(masked, axis=0, keepdims=True)
    c2 = jnp.where(masked == m2, iota, _E)
    i2 = jnp.min(c2, axis=0, keepdims=True)

    idx_ref[...] = jnp.concatenate([i1, i2], axis=0)
    denom = m1 + m2
    wts_ref[...] = jnp.concatenate([m1 / denom, m2 / denom], axis=0)


def kernel(hidden_states, gate_w):
    b, s, h = hidden_states.shape
    n = b * s
    x = hidden_states.reshape(n, h)
    grid = (n // _MB,)
    probs_t, idx_t, wts_t = pl.pallas_call(
        _router_body,
        grid=grid,
        in_specs=[
            pl.BlockSpec((_MB, h), lambda i: (i, 0)),
            pl.BlockSpec((_E, h), lambda i: (0, 0)),
        ],
        out_specs=[
            pl.BlockSpec((_E, _MB), lambda i: (0, i)),
            pl.BlockSpec((_TOPK, _MB), lambda i: (0, i)),
            pl.BlockSpec((_TOPK, _MB), lambda i: (0, i)),
        ],
        out_shape=[
            jax.ShapeDtypeStruct((_E, n), jnp.float32),
            jax.ShapeDtypeStruct((_TOPK, n), jnp.int32),
            jax.ShapeDtypeStruct((_TOPK, n), jnp.float32),
        ],
        compiler_params=pltpu.CompilerParams(
            dimension_semantics=("arbitrary",),
        ),
    )(x, gate_w)
    return (probs_t.T.reshape(b, s, _E),
            idx_t.T.reshape(b, s, _TOPK),
            wts_t.T.reshape(b, s, _TOPK))


# speedup vs baseline: 1.1904x; 1.1904x over previous
import jax
import jax.numpy as jnp
from jax.experimental import pallas as pl
from jax.experimental.pallas import tpu as pltpu

_E = 16
_TOPK = 2

_MB = 1024


def _router_body(xa_ref, xb_ref, wa_ref, wb_ref, probs_ref, idx_ref, wts_ref):
    logits = (
        jax.lax.dot_general(
            xa_ref[...], wa_ref[...], (((1,), (1,)), ((), ())),
            preferred_element_type=jnp.float32)
        + jax.lax.dot_general(
            xb_ref[...], wb_ref[...], (((1,), (1,)), ((), ())),
            preferred_element_type=jnp.float32))
    lt = logits.T
    m = jnp.max(lt, axis=0, keepdims=True)
    e = jnp.exp(lt - m)
    s = jnp.sum(e, axis=0, keepdims=True)
    p = e / s
    probs_ref[...] = p

    iota = jax.lax.broadcasted_iota(jnp.int32, p.shape, 0)
    m1 = jnp.max(p, axis=0, keepdims=True)
    c1 = jnp.where(p == m1, iota, _E)
    i1 = jnp.min(c1, axis=0, keepdims=True)
    masked = jnp.where(iota == i1, -1.0, p)
    m2 = jnp.max(masked, axis=0, keepdims=True)
    c2 = jnp.where(masked == m2, iota, _E)
    i2 = jnp.min(c2, axis=0, keepdims=True)

    idx_ref[...] = jnp.concatenate([i1, i2], axis=0)
    denom = m1 + m2
    wts_ref[...] = jnp.concatenate([m1 / denom, m2 / denom], axis=0)


def kernel(hidden_states, gate_w):
    b, s, h = hidden_states.shape
    n = b * s
    hh = h // 2
    x = hidden_states.reshape(n, h)
    grid = (n // _MB,)
    probs_t, idx_t, wts_t = pl.pallas_call(
        _router_body,
        grid=grid,
        in_specs=[
            pl.BlockSpec((_MB, hh), lambda i: (i, 0)),
            pl.BlockSpec((_MB, hh), lambda i: (i, 1)),
            pl.BlockSpec((_E, hh), lambda i: (0, 0)),
            pl.BlockSpec((_E, hh), lambda i: (0, 1)),
        ],
        out_specs=[
            pl.BlockSpec((_E, _MB), lambda i: (0, i)),
            pl.BlockSpec((_TOPK, _MB), lambda i: (0, i)),
            pl.BlockSpec((_TOPK, _MB), lambda i: (0, i)),
        ],
        out_shape=[
            jax.ShapeDtypeStruct((_E, n), jnp.float32),
            jax.ShapeDtypeStruct((_TOPK, n), jnp.int32),
            jax.ShapeDtypeStruct((_TOPK, n), jnp.float32),
        ],
        compiler_params=pltpu.CompilerParams(
            dimension_semantics=("arbitrary",),
        ),
    )(x, x, gate_w, gate_w)
    return (probs_t.T.reshape(b, s, _E),
            idx_t.T.reshape(b, s, _TOPK),
            wts_t.T.reshape(b, s, _TOPK))
